# Initial kernel scaffold; baseline (speedup 1.0000x reference)
import jax
import jax.numpy as jnp
from jax.experimental import pallas as pl


def kernel(x, edge_index, pos, batch, W1, b1, W2, b2, W3, b3, Wt, bt):
    raise NotImplementedError("write your pallas kernel here")


# speedup vs baseline: 2.8822x; 2.8822x over previous
import functools

import jax
import jax.numpy as jnp
from jax import lax
from jax.experimental import pallas as pl
from jax.experimental.pallas import tpu as pltpu
from jax.experimental.pallas import tpu_sc as plsc

N = 10000
NP = 10240
D = 256
DH = 128
BN = 256
NBLK = NP // BN
E_CHUNK = 128
PAD_B = 1e6


def _layer_body(relu_in, scale_in, scale_out, hin_ref, w_ref, b_ref,
                dinv_ref, o0_ref, o1_ref):
    h = hin_ref[...]
    dv = dinv_ref[...]
    if scale_in:
        h = h * dv + b_ref[...]
        if relu_in:
            h = jnp.maximum(h, 0.0)
    z = jnp.dot(h, w_ref[...], preferred_element_type=jnp.float32)
    if scale_out:
        z = z * dv
    o0_ref[...] = z[:, :DH]
    o1_ref[...] = z[:, DH:]


def _layer(hin, w, b_prev, dinv, relu_in, scale_in, scale_out):
    body = functools.partial(_layer_body, relu_in, scale_in, scale_out)
    return pl.pallas_call(
        body,
        grid=(NBLK,),
        in_specs=[
            pl.BlockSpec((BN, D), lambda i: (i, 0)),
            pl.BlockSpec((D, D), lambda i: (0, 0)),
            pl.BlockSpec((1, D), lambda i: (0, 0)),
            pl.BlockSpec((BN, 1), lambda i: (i, 0)),
        ],
        out_specs=[
            pl.BlockSpec((BN, DH), lambda i: (i, 0)),
            pl.BlockSpec((BN, DH), lambda i: (i, 0)),
        ],
        out_shape=[
            jax.ShapeDtypeStruct((NP, DH), jnp.float32),
            jax.ShapeDtypeStruct((NP, DH), jnp.float32),
        ],
    )(hin, w, b_prev, dinv)


def _final_body(s_ref, b3_ref, dinv_ref, wt_ref, bt_ref, pos_ref,
                h_ref, tp_ref):
    h = s_ref[...] * dinv_ref[...] + b3_ref[...]
    h_ref[...] = h
    off = jnp.dot(h, wt_ref[...], preferred_element_type=jnp.float32)
    tp_ref[...] = pos_ref[...] + off + bt_ref[...]


def _final_layer(s3, b3, dinv, wt, bt, pos_pad):
    return pl.pallas_call(
        _final_body,
        grid=(NBLK,),
        in_specs=[
            pl.BlockSpec((BN, D), lambda i: (i, 0)),
            pl.BlockSpec((1, D), lambda i: (0, 0)),
            pl.BlockSpec((BN, 1), lambda i: (i, 0)),
            pl.BlockSpec((D, 3), lambda i: (0, 0)),
            pl.BlockSpec((1, 3), lambda i: (0, 0)),
            pl.BlockSpec((BN, 3), lambda i: (i, 0)),
        ],
        out_specs=[
            pl.BlockSpec((BN, D), lambda i: (i, 0)),
            pl.BlockSpec((BN, 3), lambda i: (i, 0)),
        ],
        out_shape=[
            jax.ShapeDtypeStruct((NP, D), jnp.float32),
            jax.ShapeDtypeStruct((NP, 3), jnp.float32),
        ],
    )(s3, b3, dinv, wt, bt, pos_pad)


def _make_sc_segsum(ep_total):
    per_sub = ep_total // 16
    n_chunks = per_sub // E_CHUNK
    rows_per_sub = NP // 16

    mesh = plsc.VectorSubcoreMesh(core_axis_name="c", subcore_axis_name="s")

    @functools.partial(
        pl.kernel, mesh=mesh,
        out_type=[
            jax.ShapeDtypeStruct((NP, DH), jnp.float32),
            jax.ShapeDtypeStruct((NP, DH), jnp.float32),
        ],
        scratch_types=[
            pltpu.VMEM((E_CHUNK,), jnp.int32),
            pltpu.VMEM((E_CHUNK,), jnp.int32),
            pltpu.VMEM((E_CHUNK, DH), jnp.float32),
            pltpu.VMEM_SHARED((NP, DH), jnp.float32),
            pltpu.SemaphoreType.DMA,
        ],
    )
    def segsum(z0_hbm, z1_hbm, src_hbm, dst_hbm, s0_hbm, s1_hbm,
               sidx_v, didx_v, rows_v, shared, sem):
        cid = lax.axis_index("c")
        sid = lax.axis_index("s")
        row0 = sid * rows_per_sub

        def half(z_hbm, out_hbm):
            pltpu.sync_copy(z_hbm.at[pl.ds(row0, rows_per_sub)],
                            shared.at[pl.ds(row0, rows_per_sub)])
            plsc.subcore_barrier()
            def body(k, _):
                base = sid * per_sub + k * E_CHUNK
                pltpu.sync_copy(src_hbm.at[pl.ds(base, E_CHUNK)], sidx_v)
                pltpu.sync_copy(dst_hbm.at[pl.ds(base, E_CHUNK)], didx_v)
                pltpu.async_copy(z_hbm.at[sidx_v], rows_v, sem).wait()
                pltpu.sync_copy(rows_v, shared.at[didx_v], add=True)
                return 0
            lax.fori_loop(0, n_chunks, body, 0)
            plsc.subcore_barrier()
            pltpu.sync_copy(shared.at[pl.ds(row0, rows_per_sub)],
                            out_hbm.at[pl.ds(row0, rows_per_sub)])

        @pl.when(cid == 0)
        def _():
            half(z0_hbm, s0_hbm)

        @pl.when(cid == 1)
        def _():
            half(z1_hbm, s1_hbm)

    return segsum


def _nn_body(a_ref, bt_ref, idx_ref):
    a = a_ref[...]
    bt = bt_ref[...]
    ab = jnp.dot(a, bt, preferred_element_type=jnp.float32)
    asq = jnp.sum(a * a, axis=1, keepdims=True)
    bsq = jnp.sum(bt * bt, axis=0, keepdims=True)
    d = asq + bsq - 2.0 * ab
    m = jnp.min(d, axis=1, keepdims=True)
    ii = lax.broadcasted_iota(jnp.int32, (BN, NP), 1)
    sel = jnp.where(d == m, ii, jnp.int32(2 ** 30))
    idx_ref[0, 0, :] = jnp.min(sel, axis=1)


def _nn_argmin(a_pad, bT_pad):
    out = pl.pallas_call(
        _nn_body,
        grid=(NBLK,),
        in_specs=[
            pl.BlockSpec((BN, 3), lambda i: (i, 0)),
            pl.BlockSpec((3, NP), lambda i: (0, 0)),
        ],
        out_specs=pl.BlockSpec((1, 1, BN), lambda i: (i, 0, 0)),
        out_shape=jax.ShapeDtypeStruct((NBLK, 1, BN), jnp.int32),
    )(a_pad, bT_pad)
    return out.reshape(NP)[:N]


def _pad_rows(x, total, value=0.0):
    return jnp.pad(x, ((0, total - x.shape[0]), (0, 0)),
                   constant_values=value)


def kernel(x, edge_index, pos, batch, W1, b1, W2, b2, W3, b3, Wt, bt):
    n = x.shape[0]
    src = edge_index[0]
    dst = edge_index[1]

    deg = jnp.zeros((n,), jnp.float32).at[dst].add(1.0) + 1.0
    dinv = lax.rsqrt(deg)
    dinv_pad = jnp.concatenate(
        [dinv, jnp.ones((NP - n,), jnp.float32)]).reshape(NP, 1)

    e = src.shape[0]
    ep = ((e + 16 * E_CHUNK - 1) // (16 * E_CHUNK)) * (16 * E_CHUNK)
    pad_e = ep - e
    src_p = jnp.concatenate(
        [src, jnp.full((pad_e,), NP - 1, jnp.int32)]).astype(jnp.int32)
    dst_p = jnp.concatenate(
        [dst, jnp.full((pad_e,), NP - 1, jnp.int32)]).astype(jnp.int32)

    segsum = _make_sc_segsum(ep)

    x_pad = _pad_rows(x, NP)
    pos_pad = _pad_rows(pos, NP)
    zero_b = jnp.zeros((1, D), jnp.float32)

    z0, z1 = _layer(x_pad, W1, zero_b, dinv_pad,
                    relu_in=False, scale_in=False, scale_out=True)
    s0, s1 = segsum(z0, z1, src_p, dst_p)
    s_1 = jnp.concatenate([s0, s1], axis=1)

    z0, z1 = _layer(s_1, W2, b1.reshape(1, D), dinv_pad,
                    relu_in=True, scale_in=True, scale_out=True)
    s0, s1 = segsum(z0, z1, src_p, dst_p)
    s_2 = jnp.concatenate([s0, s1], axis=1)

    z0, z1 = _layer(s_2, W3, b2.reshape(1, D), dinv_pad,
                    relu_in=True, scale_in=True, scale_out=True)
    s0, s1 = segsum(z0, z1, src_p, dst_p)
    s_3 = jnp.concatenate([s0, s1], axis=1)

    h_pad, tp_pad = _final_layer(s_3, b3.reshape(1, D), dinv_pad,
                                 Wt, bt.reshape(1, 3), pos_pad)
    h = h_pad[:n]
    transformed_pos = tp_pad[:n]

    tgt = pos
    bT_tgt = jnp.pad(tgt, ((0, NP - n), (0, 0)),
                     constant_values=PAD_B).T
    cur = transformed_pos
    for _ in range(10):
        cur_pad = _pad_rows(cur, NP)
        idx = _nn_argmin(cur_pad, bT_tgt)
        corr = tgt[idx]
        mu_s = cur.mean(axis=0)
        mu_t = corr.mean(axis=0)
        cs = cur - mu_s
        ct = corr - mu_t
        H = cs.T @ ct
        U, S, Vt = jnp.linalg.svd(H)
        sgn = jnp.sign(jnp.linalg.det(Vt.T @ U.T))
        Dm = jnp.diag(jnp.concatenate([jnp.ones((2,), H.dtype), sgn[None]]))
        R = Vt.T @ Dm @ U.T
        t = mu_t - R @ mu_s
        cur = cur @ R.T + t
    aligned = cur

    a_pad = _pad_rows(aligned, NP)
    ia = _nn_argmin(a_pad, bT_tgt)
    bT_al = jnp.pad(aligned, ((0, NP - n), (0, 0)),
                    constant_values=PAD_B).T
    ib = _nn_argmin(pos_pad, bT_al)
    da = ((aligned - tgt[ia]) ** 2).sum(1)
    db = ((tgt - aligned[ib]) ** 2).sum(1)
    loss = da.mean() + db.mean()

    return (h, aligned, loss)
